# Initial kernel scaffold; baseline (speedup 1.0000x reference)
import jax
import jax.numpy as jnp
from jax.experimental import pallas as pl


def kernel(token_ids, weight):
    raise NotImplementedError("write your pallas kernel here")


# speedup vs baseline: 1.5757x; 1.5757x over previous
import functools

import jax
import jax.numpy as jnp
from jax import lax
from jax.experimental import pallas as pl
from jax.experimental.pallas import tpu as pltpu
from jax.experimental.pallas import tpu_sc as plsc

_D = 64


@functools.partial(jax.jit, static_argnums=(2,))
def _gather_rows(idx, table, B):
    info = plsc.get_sparse_core_info()
    nw = info.num_cores * info.num_subcores
    chunk = 128
    b_per_w = B // nw
    n_chunks = b_per_w // chunk
    mesh = plsc.VectorSubcoreMesh(core_axis_name="c", subcore_axis_name="s")

    @functools.partial(
        pl.kernel,
        mesh=mesh,
        out_type=jax.ShapeDtypeStruct((B, _D), jnp.float32),
        scratch_types=[
            pltpu.VMEM((chunk,), jnp.int32),
            pltpu.VMEM((chunk, _D), jnp.float32),
            pltpu.SemaphoreType.DMA,
        ],
        compiler_params=pltpu.CompilerParams(use_tc_tiling_on_sc=False),
    )
    def k(idx_hbm, table_hbm, out_hbm, idx_v, rows_v, sem):
        wid = lax.axis_index("s") * info.num_cores + lax.axis_index("c")
        base = wid * b_per_w

        def body(i, carry):
            off = base + i * chunk
            pltpu.sync_copy(idx_hbm.at[pl.ds(off, chunk)], idx_v)
            pltpu.async_copy(table_hbm.at[idx_v], rows_v, sem).wait()
            pltpu.sync_copy(rows_v, out_hbm.at[pl.ds(off, chunk)])
            return carry

        lax.fori_loop(0, n_chunks, body, 0)

    return k(idx, table)


def kernel(token_ids, weight):
    s0, s1 = token_ids.shape
    b = s0 * s1
    idx = token_ids.reshape(b).astype(jnp.int32)
    out = _gather_rows(idx, weight, b)
    return out.reshape(s0, s1, _D)


# speedup vs baseline: 1.8691x; 1.1862x over previous
import functools

import jax
import jax.numpy as jnp
from jax import lax
from jax.experimental import pallas as pl
from jax.experimental.pallas import tpu as pltpu
from jax.experimental.pallas import tpu_sc as plsc

_D = 64


@functools.partial(jax.jit, static_argnums=(2,))
def _gather_rows(idx, table, B):
    info = plsc.get_sparse_core_info()
    nw = info.num_cores * info.num_subcores
    chunk = 128
    nbuf = 4
    b_per_w = B // nw
    n_chunks = b_per_w // chunk
    n_outer = n_chunks // nbuf
    mesh = plsc.VectorSubcoreMesh(core_axis_name="c", subcore_axis_name="s")

    @functools.partial(
        pl.kernel,
        mesh=mesh,
        out_type=jax.ShapeDtypeStruct((B, _D), jnp.float32),
        scratch_types=[
            pltpu.VMEM((b_per_w,), jnp.int32),
            pltpu.VMEM((nbuf, chunk, _D), jnp.float32),
            pltpu.SemaphoreType.DMA((nbuf,)),
            pltpu.SemaphoreType.DMA((nbuf,)),
        ],
        compiler_params=pltpu.CompilerParams(use_tc_tiling_on_sc=False),
    )
    def k(idx_hbm, table_hbm, out_hbm, idx_v, rows_v, gsem, wsem):
        wid = lax.axis_index("s") * info.num_cores + lax.axis_index("c")
        base = wid * b_per_w
        pltpu.sync_copy(idx_hbm.at[pl.ds(base, b_per_w)], idx_v)

        def start_gather(i, b):
            pltpu.async_copy(
                table_hbm.at[idx_v.at[pl.ds(i * chunk, chunk)]],
                rows_v.at[b],
                gsem.at[b],
            )

        def wait_gather(i, b):
            pltpu.make_async_copy(
                table_hbm.at[idx_v.at[pl.ds(i * chunk, chunk)]],
                rows_v.at[b],
                gsem.at[b],
            ).wait()

        def start_write(i, b):
            pltpu.async_copy(
                rows_v.at[b],
                out_hbm.at[pl.ds(base + i * chunk, chunk)],
                wsem.at[b],
            )

        def wait_write(i, b):
            pltpu.make_async_copy(
                rows_v.at[b],
                out_hbm.at[pl.ds(base + i * chunk, chunk)],
                wsem.at[b],
            ).wait()

        for b in range(nbuf):
            start_gather(b, b)

        def body(o, carry):
            i0 = o * nbuf
            for b in range(nbuf):
                wait_gather(i0 + b, b)
                start_write(i0 + b, b)
            for b in range(nbuf):
                wait_write(i0 + b, b)
                start_gather(i0 + nbuf + b, b)
            return carry

        lax.fori_loop(0, n_outer - 1, body, 0, unroll=False)

        i0 = (n_outer - 1) * nbuf
        for b in range(nbuf):
            wait_gather(i0 + b, b)
            start_write(i0 + b, b)
        for b in range(nbuf):
            wait_write(i0 + b, b)

    return k(idx, table)


def kernel(token_ids, weight):
    s0, s1 = token_ids.shape
    b = s0 * s1
    idx = token_ids.reshape(b).astype(jnp.int32)
    out = _gather_rows(idx, weight, b)
    return out.reshape(s0, s1, _D)
